# Initial kernel scaffold; baseline (speedup 1.0000x reference)
import jax
import jax.numpy as jnp
from jax.experimental import pallas as pl


def kernel(x, x2, batch, batch2, y, W0, g0, b0, W1, g1, b1, Wc, gc, bc, Wm, bm, gm, bbm, Wo, bo):
    raise NotImplementedError("write your pallas kernel here")


# speedup vs baseline: 1.1927x; 1.1927x over previous
import functools

import jax
import jax.numpy as jnp
from jax.experimental import pallas as pl
from jax.experimental.pallas import tpu as pltpu

NUM_CLASS = 5
NX = 1
NY = 1
SCALE_XY = 1
SCALE_Y = 1
VX = 6.0
VY = 6.0
XOFF = -3.0
YOFF = -3.0
ZOFF = 0.0
EPS = 1e-3
N_PTS = 65536
P = 16
S = 17
SR = 32
NB = 4096
NBLK = N_PTS // NB
NEG_INF = float("-inf")

_XDOT = functools.partial(
    jax.lax.dot_general, precision=jax.lax.Precision.HIGHEST,
    preferred_element_type=jnp.float32)


def _bdot(a, b, dims):
    return jax.lax.dot_general(
        a.astype(jnp.bfloat16), b.astype(jnp.bfloat16), dims,
        preferred_element_type=jnp.float32)


def _point_features(x_ref, bt_ref):
    x6 = x_ref[0]
    px = x6[:, 0:1]
    py = x6[:, 1:2]
    pz = x6[:, 2:3]
    c0 = jnp.floor((px - XOFF) / VX)
    c1 = jnp.floor((py - YOFF) / VY)
    mask = (c0 >= 0.0) & (c0 < float(NX)) & (c1 >= 0.0) & (c1 < float(NY))
    fc = jnp.concatenate(
        [px - (c0 * VX + XOFF), py - (c1 * VY + YOFF), pz - ZOFF], axis=1)
    b = bt_ref[0, 0, :][:, None]
    merge = (b * SCALE_XY + c0.astype(jnp.int32) * SCALE_Y
             + c1.astype(jnp.int32))
    seg = jnp.where(mask, merge, S - 1)
    cols = jax.lax.broadcasted_iota(jnp.int32, (x6.shape[0], SR), 1)
    onehot = (cols == seg).astype(jnp.float32)
    return x6, fc, seg, onehot


def _mean_table(sa):
    sum_pts = sa[:, 0:3]
    cnt = sa[:, 3:4]
    nz = cnt > 0.0
    mean = jnp.where(nz, sum_pts / jnp.maximum(cnt, 1.0), 0.0)
    return mean, cnt, nz


def _raw0(x6, fc, onehot, mean, w0):
    mseg = _XDOT(onehot, mean, (((1,), (0,)), ((), ())))
    pts = x6[:, 0:3]
    hcat = jnp.concatenate([pts, pts - mseg, fc, x6[:, 3:6]], axis=1)
    return _bdot(hcat, w0, (((1,), (1,)), ((), ())))


def _bn_coeffs(total, total_sq, g, b, n):
    m = total / n
    v = total_sq / n - m * m
    a = g / jnp.sqrt(v + EPS)
    return a, b - a * m


def _pass_a_kernel(x_ref, bt_ref, sa_ref):
    i = pl.program_id(1)

    @pl.when(i == 0)
    def _():
        sa_ref[...] = jnp.zeros_like(sa_ref)

    x6, _, _, onehot = _point_features(x_ref, bt_ref)
    ones = jnp.ones((x6.shape[0], 1), jnp.float32)
    vals = jnp.concatenate([x6[:, 0:3], ones], axis=1)
    sa_ref[0, :, 0:4] += _XDOT(onehot, vals, (((0,), (0,)), ((), ())))


def _pass_b_kernel(x_ref, bt_ref, sa_ref, w0_ref, sb_ref, mean_s):
    i = pl.program_id(1)

    @pl.when(i == 0)
    def _():
        mean, _, _ = _mean_table(sa_ref[0])
        mean_s[...] = mean
        sb_ref[...] = jnp.zeros_like(sb_ref)

    x6, fc, _, onehot = _point_features(x_ref, bt_ref)
    r0 = _raw0(x6, fc, onehot, mean_s[...], w0_ref[...])
    vals = jnp.concatenate([r0, r0 * r0], axis=1)
    sb_ref[0, :, 0:64] += _XDOT(onehot, vals, (((0,), (0,)), ((), ())))


def _pass_c_kernel(x_ref, bt_ref, sa_ref, sb_ref, w0_ref, g0_ref, b0_ref,
                   w1_ref, sc_ref, mean_s, a0_s, c0_s):
    i = pl.program_id(1)

    @pl.when(i == 0)
    def _():
        mean, _, _ = _mean_table(sa_ref[0])
        mean_s[...] = mean
        a0, c0b = _bn_coeffs(jnp.sum(sb_ref[0, :, 0:32], axis=0),
                             jnp.sum(sb_ref[0, :, 32:64], axis=0),
                             g0_ref[0], b0_ref[0], float(N_PTS))
        a0_s[...] = a0[None, :]
        c0_s[...] = c0b[None, :]
        sc_ref[...] = jnp.zeros_like(sc_ref)
        sc_ref[0, :, 0:32] = jnp.full((SR, 32), NEG_INF, jnp.float32)

    x6, fc, seg, onehot = _point_features(x_ref, bt_ref)
    r0 = _raw0(x6, fc, onehot, mean_s[...], w0_ref[...])
    h = jnp.maximum(a0_s[...] * r0 + c0_s[...], 0.0)
    q = _bdot(h, w1_ref[:, 0:32], (((1,), (1,)), ((), ())))
    sc_ref[0, :, 32:96] += _XDOT(onehot, q, (((0,), (0,)), ((), ())))
    sc_ref[0, :, 96:160] += _XDOT(onehot, q * q, (((0,), (0,)), ((), ())))
    for s in range(S):
        m = jnp.max(jnp.where(seg == s, h, NEG_INF), axis=0, keepdims=True)
        sc_ref[0, s:s + 1, 0:32] = jnp.maximum(sc_ref[0, s:s + 1, 0:32], m)


def _pass_d_kernel(x_ref, bt_ref, sa_ref, sb_ref, sc_ref, w0_ref, g0_ref,
                   b0_ref, w1_ref, g1_ref, b1_ref, pill_ref, mean_s, a0_s,
                   c0_s, a1_s, bias1_s):
    i = pl.program_id(1)

    @pl.when(i == 0)
    def _():
        mean, cnt, nz = _mean_table(sa_ref[0])
        mean_s[...] = mean
        a0, c0b = _bn_coeffs(jnp.sum(sb_ref[0, :, 0:32], axis=0),
                             jnp.sum(sb_ref[0, :, 32:64], axis=0),
                             g0_ref[0], b0_ref[0], float(N_PTS))
        a0_s[...] = a0[None, :]
        c0_s[...] = c0b[None, :]
        hmax = jnp.where(nz, sc_ref[0, :, 0:32], 0.0)
        sum_q = sc_ref[0, :, 32:96]
        sum_q2 = sc_ref[0, :, 96:160]
        t = _bdot(hmax, w1_ref[:, 32:64], (((1,), (1,)), ((), ())))
        t = jnp.where(nz, t, 0.0)
        n = float(N_PTS)
        m1 = (jnp.sum(sum_q, axis=0) + jnp.sum(cnt * t, axis=0)) / n
        ex2 = (jnp.sum(sum_q2, axis=0) + 2.0 * jnp.sum(sum_q * t, axis=0)
               + jnp.sum(cnt * t * t, axis=0)) / n
        v1 = ex2 - m1 * m1
        a1 = g1_ref[0] / jnp.sqrt(v1 + EPS)
        a1_s[...] = a1[None, :]
        bias1_s[...] = a1[None, :] * t + (b1_ref[0] - a1 * m1)[None, :]
        pill_ref[...] = jnp.full(pill_ref.shape, NEG_INF, jnp.float32)

    x6, fc, seg, onehot = _point_features(x_ref, bt_ref)
    r0 = _raw0(x6, fc, onehot, mean_s[...], w0_ref[...])
    h = jnp.maximum(a0_s[...] * r0 + c0_s[...], 0.0)
    q = _bdot(h, w1_ref[:, 0:32], (((1,), (1,)), ((), ())))
    h1 = jnp.maximum(
        a1_s[...] * q + _XDOT(onehot, bias1_s[...], (((1,), (0,)), ((), ()))),
        0.0)
    for s in range(P):
        m = jnp.max(jnp.where(seg == s, h1, NEG_INF), axis=0, keepdims=True)
        pill_ref[0, s:s + 1, :] = jnp.maximum(pill_ref[0, s:s + 1, :], m)


def _bn_rows(z, g, b):
    m = jnp.mean(z, axis=0, keepdims=True)
    v = jnp.mean((z - m) * (z - m), axis=0, keepdims=True)
    return g * (z - m) / jnp.sqrt(v + EPS) + b


def _head_kernel(pill_ref, wc_ref, gc_ref, bc_ref, wm_ref, bm_ref, gm_ref,
                 bbm_ref, wo_ref, bo_ref, out_ref):
    p1 = pill_ref[0, 0:P, :]
    p2 = pill_ref[1, 0:P, :]
    z1 = _bdot(p1, wc_ref[...], (((1,), (1,)), ((), ())))
    z2 = _bdot(p2, wc_ref[...], (((1,), (1,)), ((), ())))
    z1 = jnp.maximum(_bn_rows(z1, gc_ref[...], bc_ref[...]), 0.0)
    z2 = jnp.maximum(_bn_rows(z2, gc_ref[...], bc_ref[...]), 0.0)
    d = z2 - z1
    r = _bdot(d, wm_ref[...], (((1,), (1,)), ((), ()))) + bm_ref[...]
    r = jnp.maximum(_bn_rows(r, gm_ref[...], bbm_ref[...]), 0.0)
    o = _bdot(r, wo_ref[...], (((1,), (1,)), ((), ()))) + bo_ref[...]
    colmask = jax.lax.broadcasted_iota(jnp.int32, o.shape, 1) < NUM_CLASS
    om = jnp.where(colmask, o, NEG_INF)
    mx = jnp.max(om, axis=1, keepdims=True)
    lse = jnp.log(jnp.sum(jnp.exp(om - mx), axis=1, keepdims=True)) + mx
    out_ref[...] = jnp.zeros(out_ref.shape, jnp.float32)
    out_ref[:, 0:8] = om - lse


def kernel(x, x2, batch, batch2, y, W0, g0, b0, W1, g1, b1, Wc, gc, bc, Wm,
           bm, gm, bbm, Wo, bo):
    del y
    f32 = jnp.float32
    X = jnp.stack([x, x2]).astype(f32)
    BT = jnp.stack([batch, batch2]).astype(jnp.int32).reshape(2, 1, N_PTS)
    g0r = g0.reshape(1, 32).astype(f32)
    b0r = b0.reshape(1, 32).astype(f32)
    g1r = g1.reshape(1, 64).astype(f32)
    b1r = b1.reshape(1, 64).astype(f32)
    gcr = gc.reshape(1, 1024).astype(f32)
    bcr = bc.reshape(1, 1024).astype(f32)
    bmr = bm.reshape(1, 64).astype(f32)
    gmr = gm.reshape(1, 64).astype(f32)
    bbmr = bbm.reshape(1, 64).astype(f32)
    wop = jnp.zeros((8, 64), f32).at[0:NUM_CLASS, :].set(Wo.astype(f32))
    bop = jnp.zeros((1, 8), f32).at[0, 0:NUM_CLASS].set(bo.astype(f32))

    x_spec = pl.BlockSpec((1, NB, 6), lambda b, i: (b, i, 0))
    bt_spec = pl.BlockSpec((1, 1, NB), lambda b, i: (b, 0, i))
    full = lambda shape: pl.BlockSpec(shape, lambda b, i: (0,) * len(shape))
    acc_spec = lambda c: pl.BlockSpec((1, SR, c), lambda b, i: (b, 0, 0))
    grid = (2, NBLK)

    sa = pl.pallas_call(
        _pass_a_kernel,
        grid=grid,
        in_specs=[x_spec, bt_spec],
        out_specs=acc_spec(128),
        out_shape=jax.ShapeDtypeStruct((2, SR, 128), f32),
    )(X, BT)

    sb = pl.pallas_call(
        _pass_b_kernel,
        grid=grid,
        in_specs=[x_spec, bt_spec, acc_spec(128), full((32, 12))],
        out_specs=acc_spec(128),
        out_shape=jax.ShapeDtypeStruct((2, SR, 128), f32),
        scratch_shapes=[pltpu.VMEM((SR, 3), f32)],
    )(X, BT, sa, W0)

    sc = pl.pallas_call(
        _pass_c_kernel,
        grid=grid,
        in_specs=[x_spec, bt_spec, acc_spec(128), acc_spec(128),
                  full((32, 12)), full((1, 32)), full((1, 32)),
                  full((64, 64))],
        out_specs=acc_spec(256),
        out_shape=jax.ShapeDtypeStruct((2, SR, 256), f32),
        scratch_shapes=[pltpu.VMEM((SR, 3), f32), pltpu.VMEM((1, 32), f32),
                        pltpu.VMEM((1, 32), f32)],
    )(X, BT, sa, sb, W0, g0r, b0r, W1)

    pill = pl.pallas_call(
        _pass_d_kernel,
        grid=grid,
        in_specs=[x_spec, bt_spec, acc_spec(128), acc_spec(128),
                  acc_spec(256), full((32, 12)), full((1, 32)),
                  full((1, 32)), full((64, 64)), full((1, 64)),
                  full((1, 64))],
        out_specs=acc_spec(64),
        out_shape=jax.ShapeDtypeStruct((2, SR, 64), f32),
        scratch_shapes=[pltpu.VMEM((SR, 3), f32), pltpu.VMEM((1, 32), f32),
                        pltpu.VMEM((1, 32), f32), pltpu.VMEM((1, 64), f32),
                        pltpu.VMEM((SR, 64), f32)],
    )(X, BT, sa, sb, sc, W0, g0r, b0r, W1, g1r, b1r)

    out = pl.pallas_call(
        _head_kernel,
        in_specs=[pl.BlockSpec((2, SR, 64), lambda: (0, 0, 0)),
                  pl.BlockSpec((1024, 64), lambda: (0, 0)),
                  pl.BlockSpec((1, 1024), lambda: (0, 0)),
                  pl.BlockSpec((1, 1024), lambda: (0, 0)),
                  pl.BlockSpec((64, 1024), lambda: (0, 0)),
                  pl.BlockSpec((1, 64), lambda: (0, 0)),
                  pl.BlockSpec((1, 64), lambda: (0, 0)),
                  pl.BlockSpec((1, 64), lambda: (0, 0)),
                  pl.BlockSpec((8, 64), lambda: (0, 0)),
                  pl.BlockSpec((1, 8), lambda: (0, 0))],
        out_specs=pl.BlockSpec((P, 128), lambda: (0, 0)),
        out_shape=jax.ShapeDtypeStruct((P, 128), f32),
    )(pill, Wc, gcr, bcr, Wm, bmr, gmr, bbmr, wop, bop)

    return out[:, :NUM_CLASS]
